# baseline (speedup 1.0000x reference)
import functools

import jax
import jax.numpy as jnp
from jax import lax
from jax.experimental import pallas as pl
from jax.experimental.pallas import tpu as pltpu
from jax.experimental.pallas import tpu_sc as plsc

NC = 2
NS = 16
L = 16
NW = NC * NS

N_ROWS = 16384
W = 128
RPW = N_ROWS // NW
CH = 128
NCH = RPW // CH
GPC = CH // L
NU = 16
NB = 24


def _splat(vec, i):
    return vec.at[jnp.full((L,), i, dtype=jnp.int32)].get(
        mode="promise_in_bounds")


def _sc_body(t_hbm, uv_hbm, w0_hbm, w1_hbm, bv1_hbm, bv2_hbm,
             w00_hbm, w01_hbm, w10_hbm, w11_hbm, out_hbm,
             xv, accv, uvv, w0v, w1v, bv1v, bv2v, w00v, w01v, w10v,
             w11v, sems, csem):
    wid = lax.axis_index("s") * NC + lax.axis_index("c")
    base = wid * RPW
    lane = lax.iota(jnp.int32, L)

    xcopies = [
        pltpu.async_copy(t_hbm.at[pl.ds((base + j * CH) * W, CH * W)],
                         xv.at[pl.ds(j * CH * W, CH * W)], sems[j])
        for j in range(NCH)
    ]
    small = [
        pltpu.async_copy(uv_hbm, uvv, csem),
        pltpu.async_copy(w0_hbm, w0v, csem),
        pltpu.async_copy(w1_hbm, w1v, csem),
        pltpu.async_copy(bv1_hbm, bv1v, csem),
        pltpu.async_copy(bv2_hbm, bv2v, csem),
        pltpu.async_copy(w00_hbm, w00v, csem),
        pltpu.async_copy(w01_hbm, w01v, csem),
        pltpu.async_copy(w10_hbm, w10v, csem),
        pltpu.async_copy(w11_hbm, w11v, csem),
    ]
    for c in small:
        c.wait()

    uvec = uvv[...]
    avec = w1v[...] - w0v[...]

    def chunks(ref):
        return ref[pl.ds(0, L)], ref[pl.ds(8, L)]

    i1, i2 = chunks(bv1v)
    j1, j2 = chunks(bv2v)
    w00a, w00b = chunks(w00v)
    w01a, w01b = chunks(w01v)
    w10a, w10b = chunks(w10v)
    w11a, w11b = chunks(w11v)
    b1a, b1b = w10a - w00a, w10b - w00b
    b2a, b2b = w01a - w00a, w01b - w00b
    bba, bbb = w00a - w01a - w10a + w11a, w00b - w01b - w10b + w11b

    zeros = jnp.zeros((L,), jnp.float32)
    c0_parts = (w0v[...] + w00a + jnp.where(lane >= 8, w00b, zeros))
    c0 = jnp.sum(c0_parts)
    c0v = jnp.full((L,), c0, dtype=jnp.float32)

    terms_u = []
    for t in range(NU):
        terms_u.append((_splat(uvec, t), _splat(avec, t)))
    terms_b = []
    for t in range(NB):
        if t < 8:
            iv, jv, b1, b2, bb = i1, j1, b1a, b2a, bba
            e = t
        else:
            iv, jv, b1, b2, bb = i2, j2, b1b, b2b, bbb
            e = t - 8
        terms_b.append((_splat(iv, e), _splat(jv, e),
                        _splat(b1, e), _splat(b2, e), _splat(bb, e)))

    for j in range(NCH):
        xcopies[j].wait()

        def group_body(k, _, j=j):
            rowoff = (j * CH + k * L) * W + lane * W
            acc = c0v
            for (cl, at) in terms_u:
                tv = plsc.load_gather(xv, [rowoff + cl])
                acc = acc + at * tv
            for (cli, clj, b1t, b2t, bbt) in terms_b:
                t1 = plsc.load_gather(xv, [rowoff + cli])
                t2 = plsc.load_gather(xv, [rowoff + clj])
                acc = acc + t1 * (b1t + bbt * t2) + b2t * t2
            accv[pl.ds(j * CH + k * L, L)] = acc
            return _

        lax.fori_loop(0, GPC, group_body, None)

    pltpu.sync_copy(accv, out_hbm.at[pl.ds(base, RPW)])


@functools.partial(
    pl.kernel,
    out_type=jax.ShapeDtypeStruct((N_ROWS,), jnp.float32),
    mesh=plsc.VectorSubcoreMesh(core_axis_name="c", subcore_axis_name="s",
                                num_cores=NC, num_subcores=NS),
    scratch_types=[
        pltpu.VMEM((RPW * W,), jnp.float32),
        pltpu.VMEM((RPW,), jnp.float32),
        pltpu.VMEM((NU,), jnp.int32),
        pltpu.VMEM((NU,), jnp.float32),
        pltpu.VMEM((NU,), jnp.float32),
        pltpu.VMEM((NB,), jnp.int32),
        pltpu.VMEM((NB,), jnp.int32),
        pltpu.VMEM((NB,), jnp.float32),
        pltpu.VMEM((NB,), jnp.float32),
        pltpu.VMEM((NB,), jnp.float32),
        pltpu.VMEM((NB,), jnp.float32),
        [pltpu.SemaphoreType.DMA for _ in range(NCH)],
        pltpu.SemaphoreType.DMA,
    ],
    compiler_params=pltpu.CompilerParams(needs_layout_passes=False),
)
def _pgm_loss_sc(t_hbm, uv_hbm, w0_hbm, w1_hbm, bv1_hbm, bv2_hbm,
                 w00_hbm, w01_hbm, w10_hbm, w11_hbm, out_hbm,
                 xv, accv, uvv, w0v, w1v, bv1v, bv2v, w00v, w01v, w10v,
                 w11v, sems, csem):
    _sc_body(t_hbm, uv_hbm, w0_hbm, w1_hbm, bv1_hbm, bv2_hbm,
             w00_hbm, w01_hbm, w10_hbm, w11_hbm, out_hbm,
             xv, accv, uvv, w0v, w1v, bv1v, bv2v, w00v, w01v, w10v,
             w11v, sems, csem)


def kernel(X, y, univariate_vars, univariate_weights_0, univariate_weights_1,
           bivariate_vars_1, bivariate_vars_2, bivariate_weights_00,
           bivariate_weights_01, bivariate_weights_10, bivariate_weights_11):
    total = jnp.concatenate((X, y), axis=1).reshape(-1)
    return _pgm_loss_sc(total, univariate_vars,
                        univariate_weights_0, univariate_weights_1,
                        bivariate_vars_1, bivariate_vars_2,
                        bivariate_weights_00, bivariate_weights_01,
                        bivariate_weights_10, bivariate_weights_11)


# speedup vs baseline: 1.5923x; 1.5923x over previous
import functools

import jax
import jax.numpy as jnp
from jax import lax
from jax.experimental import pallas as pl
from jax.experimental.pallas import tpu as pltpu
from jax.experimental.pallas import tpu_sc as plsc

NC = 2
NS = 16
L = 16
NW = NC * NS

N_ROWS = 16384
D = 127
RPW = N_ROWS // NW
G = RPW // L
U = 4
NCH = 4
CHR = RPW // NCH
NU = 16
NB = 24


def _splat(vec, i):
    return vec.at[jnp.full((L,), i, dtype=jnp.int32)].get(
        mode="promise_in_bounds")


def _sc_body(x_hbm, y_hbm, uv_hbm, w0_hbm, w1_hbm, bv1_hbm, bv2_hbm,
             w00_hbm, w01_hbm, w10_hbm, w11_hbm, out_hbm,
             xv, yv, accv, uvv, w0v, w1v, bv1v, bv2v, w00v, w01v, w10v,
             w11v, sems, csem):
    wid = lax.axis_index("s") * NC + lax.axis_index("c")
    base = wid * RPW

    xcopies = [
        pltpu.async_copy(
            x_hbm.at[pl.ds((base + j * CHR) * D, CHR * D)],
            xv.at[pl.ds(j * CHR * D, CHR * D)], sems[j])
        for j in range(NCH)
    ]
    copies = [
        pltpu.async_copy(y_hbm.at[pl.ds(base, RPW)], yv, csem),
        pltpu.async_copy(uv_hbm, uvv, csem),
        pltpu.async_copy(w0_hbm, w0v, csem),
        pltpu.async_copy(w1_hbm, w1v, csem),
        pltpu.async_copy(bv1_hbm, bv1v, csem),
        pltpu.async_copy(bv2_hbm, bv2v, csem),
        pltpu.async_copy(w00_hbm, w00v, csem),
        pltpu.async_copy(w01_hbm, w01v, csem),
        pltpu.async_copy(w10_hbm, w10v, csem),
        pltpu.async_copy(w11_hbm, w11v, csem),
    ]
    for c in copies:
        c.wait()

    lane = lax.iota(jnp.int32, L)

    uvec = uvv[...]
    avec = w1v[...] - w0v[...]

    def chunks(ref):
        return ref[pl.ds(0, L)], ref[pl.ds(8, L)]

    i1, i2 = chunks(bv1v)
    j1, j2 = chunks(bv2v)
    w00a, w00b = chunks(w00v)
    w01a, w01b = chunks(w01v)
    w10a, w10b = chunks(w10v)
    w11a, w11b = chunks(w11v)
    b1a, b1b = w10a - w00a, w10b - w00b
    b2a, b2b = w01a - w00a, w01b - w00b
    bba, bbb = w00a - w01a - w10a + w11a, w00b - w01b - w10b + w11b

    zeros = jnp.zeros((L,), jnp.float32)
    c0_parts = (w0v[...] + w00a + jnp.where(lane >= 8, w00b, zeros))
    c0 = jnp.sum(c0_parts)
    c0v = jnp.full((L,), c0, dtype=jnp.float32)

    def iter_body(g0):
        rowoffs = [(g0 + u) * (L * D) + lane * D for u in range(U)]
        ygs = [yv[pl.ds((g0 + u) * L, L)] for u in range(U)]
        accs = [c0v for _ in range(U)]

        for t in range(NU):
            idxs = _splat(uvec, t)
            cl = jnp.minimum(idxs, D - 1)
            isy = idxs == D
            at = _splat(avec, t)
            for u in range(U):
                tv = plsc.load_gather(xv, [rowoffs[u] + cl])
                tv = jnp.where(isy, ygs[u], tv)
                accs[u] = accs[u] + at * tv

        for t in range(NB):
            if t < 8:
                iv, jv = i1, j1
                b1, b2, bb = b1a, b2a, bba
                e = t
            else:
                iv, jv = i2, j2
                b1, b2, bb = b1b, b2b, bbb
                e = t - 8
            iis = _splat(iv, e)
            jjs = _splat(jv, e)
            cli = jnp.minimum(iis, D - 1)
            clj = jnp.minimum(jjs, D - 1)
            isyi = iis == D
            isyj = jjs == D
            b1t = _splat(b1, e)
            b2t = _splat(b2, e)
            bbt = _splat(bb, e)
            for u in range(U):
                t1 = plsc.load_gather(xv, [rowoffs[u] + cli])
                t1 = jnp.where(isyi, ygs[u], t1)
                t2 = plsc.load_gather(xv, [rowoffs[u] + clj])
                t2 = jnp.where(isyj, ygs[u], t2)
                accs[u] = accs[u] + t1 * (b1t + bbt * t2) + b2t * t2

        for u in range(U):
            accv[pl.ds((g0 + u) * L, L)] = accs[u]

    gpc = G // NCH
    for j in range(NCH):
        xcopies[j].wait()
        lax.fori_loop(0, gpc // U,
                      lambda it, _, j=j: iter_body(j * gpc + it * U), None)

    pltpu.sync_copy(accv, out_hbm.at[pl.ds(base, RPW)])


@functools.partial(
    pl.kernel,
    out_type=jax.ShapeDtypeStruct((N_ROWS,), jnp.float32),
    mesh=plsc.VectorSubcoreMesh(core_axis_name="c", subcore_axis_name="s",
                                num_cores=NC, num_subcores=NS),
    scratch_types=[
        pltpu.VMEM((RPW * D,), jnp.float32),
        pltpu.VMEM((RPW,), jnp.float32),
        pltpu.VMEM((RPW,), jnp.float32),
        pltpu.VMEM((NU,), jnp.int32),
        pltpu.VMEM((NU,), jnp.float32),
        pltpu.VMEM((NU,), jnp.float32),
        pltpu.VMEM((NB,), jnp.int32),
        pltpu.VMEM((NB,), jnp.int32),
        pltpu.VMEM((NB,), jnp.float32),
        pltpu.VMEM((NB,), jnp.float32),
        pltpu.VMEM((NB,), jnp.float32),
        pltpu.VMEM((NB,), jnp.float32),
        [pltpu.SemaphoreType.DMA for _ in range(NCH)],
        pltpu.SemaphoreType.DMA,
    ],
    compiler_params=pltpu.CompilerParams(needs_layout_passes=False),
)
def _pgm_loss_sc(x_hbm, y_hbm, uv_hbm, w0_hbm, w1_hbm, bv1_hbm, bv2_hbm,
                 w00_hbm, w01_hbm, w10_hbm, w11_hbm, out_hbm,
                 xv, yv, accv, uvv, w0v, w1v, bv1v, bv2v, w00v, w01v, w10v,
                 w11v, sems, csem):
    _sc_body(x_hbm, y_hbm, uv_hbm, w0_hbm, w1_hbm, bv1_hbm, bv2_hbm,
             w00_hbm, w01_hbm, w10_hbm, w11_hbm, out_hbm,
             xv, yv, accv, uvv, w0v, w1v, bv1v, bv2v, w00v, w01v, w10v,
             w11v, sems, csem)


def kernel(X, y, univariate_vars, univariate_weights_0, univariate_weights_1,
           bivariate_vars_1, bivariate_vars_2, bivariate_weights_00,
           bivariate_weights_01, bivariate_weights_10, bivariate_weights_11):
    return _pgm_loss_sc(X.reshape(-1), y[:, 0], univariate_vars,
                        univariate_weights_0, univariate_weights_1,
                        bivariate_vars_1, bivariate_vars_2,
                        bivariate_weights_00, bivariate_weights_01,
                        bivariate_weights_10, bivariate_weights_11)


# speedup vs baseline: 1.6922x; 1.0627x over previous
import functools

import jax
import jax.numpy as jnp
from jax import lax
from jax.experimental import pallas as pl
from jax.experimental.pallas import tpu as pltpu
from jax.experimental.pallas import tpu_sc as plsc

NC = 2
NS = 16
L = 16
NW = NC * NS

N_ROWS = 16384
D = 127
RPW = N_ROWS // NW
G = RPW // L
U = 8
CHR = RPW // 2
NU = 16
NB = 24


def _splat(vec, i):
    return vec.at[jnp.full((L,), i, dtype=jnp.int32)].get(
        mode="promise_in_bounds")


def _sc_body(x_hbm, y_hbm, uv_hbm, w0_hbm, w1_hbm, bv1_hbm, bv2_hbm,
             w00_hbm, w01_hbm, w10_hbm, w11_hbm, out_hbm,
             xv, yv, accv, uvv, w0v, w1v, bv1v, bv2v, w00v, w01v, w10v,
             w11v, sem0, sem1, csem):
    wid = lax.axis_index("s") * NC + lax.axis_index("c")
    base = wid * RPW

    def xchunk(j, s):
        return pltpu.make_async_copy(
            x_hbm.at[pl.ds((base + j * CHR) * D, CHR * D)],
            xv.at[pl.ds(j * CHR * D, CHR * D)], s)

    xc0, xc1 = xchunk(0, sem0), xchunk(1, sem1)
    xc0.start()
    xc1.start()
    copies = [
        pltpu.async_copy(y_hbm.at[pl.ds(base, RPW)], yv, csem),
        pltpu.async_copy(uv_hbm, uvv, csem),
        pltpu.async_copy(w0_hbm, w0v, csem),
        pltpu.async_copy(w1_hbm, w1v, csem),
        pltpu.async_copy(bv1_hbm, bv1v, csem),
        pltpu.async_copy(bv2_hbm, bv2v, csem),
        pltpu.async_copy(w00_hbm, w00v, csem),
        pltpu.async_copy(w01_hbm, w01v, csem),
        pltpu.async_copy(w10_hbm, w10v, csem),
        pltpu.async_copy(w11_hbm, w11v, csem),
    ]
    for c in copies:
        c.wait()

    lane = lax.iota(jnp.int32, L)

    uvec = uvv[...]
    avec = w1v[...] - w0v[...]

    def chunks(ref):
        return ref[pl.ds(0, L)], ref[pl.ds(8, L)]

    i1, i2 = chunks(bv1v)
    j1, j2 = chunks(bv2v)
    w00a, w00b = chunks(w00v)
    w01a, w01b = chunks(w01v)
    w10a, w10b = chunks(w10v)
    w11a, w11b = chunks(w11v)
    b1a, b1b = w10a - w00a, w10b - w00b
    b2a, b2b = w01a - w00a, w01b - w00b
    bba, bbb = w00a - w01a - w10a + w11a, w00b - w01b - w10b + w11b

    zeros = jnp.zeros((L,), jnp.float32)
    c0_parts = (w0v[...] + w00a + jnp.where(lane >= 8, w00b, zeros))
    c0 = jnp.sum(c0_parts)
    c0v = jnp.full((L,), c0, dtype=jnp.float32)

    def iter_body(it, _):
        @pl.when(it == (G // U) // 2)
        def _wait_second_chunk():
            xchunk(1, sem1).wait()

        g0 = it * U
        rowoffs = [(g0 + u) * (L * D) + lane * D for u in range(U)]
        ygs = [yv[pl.ds((g0 + u) * L, L)] for u in range(U)]
        accs = [c0v for _ in range(U)]

        for t in range(NU):
            idxs = _splat(uvec, t)
            cl = jnp.minimum(idxs, D - 1)
            isy = idxs == D
            at = _splat(avec, t)
            for u in range(U):
                tv = plsc.load_gather(xv, [rowoffs[u] + cl])
                tv = jnp.where(isy, ygs[u], tv)
                accs[u] = accs[u] + at * tv

        for t in range(NB):
            if t < 8:
                iv, jv = i1, j1
                b1, b2, bb = b1a, b2a, bba
                e = t
            else:
                iv, jv = i2, j2
                b1, b2, bb = b1b, b2b, bbb
                e = t - 8
            iis = _splat(iv, e)
            jjs = _splat(jv, e)
            cli = jnp.minimum(iis, D - 1)
            clj = jnp.minimum(jjs, D - 1)
            isyi = iis == D
            isyj = jjs == D
            b1t = _splat(b1, e)
            b2t = _splat(b2, e)
            bbt = _splat(bb, e)
            for u in range(U):
                t1 = plsc.load_gather(xv, [rowoffs[u] + cli])
                t1 = jnp.where(isyi, ygs[u], t1)
                t2 = plsc.load_gather(xv, [rowoffs[u] + clj])
                t2 = jnp.where(isyj, ygs[u], t2)
                accs[u] = accs[u] + t1 * (b1t + bbt * t2) + b2t * t2

        for u in range(U):
            accv[pl.ds((g0 + u) * L, L)] = accs[u]
        return _

    xc0.wait()
    lax.fori_loop(0, G // U, iter_body, None)
    pltpu.sync_copy(accv, out_hbm.at[pl.ds(base, RPW)])


@functools.partial(
    pl.kernel,
    out_type=jax.ShapeDtypeStruct((N_ROWS,), jnp.float32),
    mesh=plsc.VectorSubcoreMesh(core_axis_name="c", subcore_axis_name="s",
                                num_cores=NC, num_subcores=NS),
    scratch_types=[
        pltpu.VMEM((RPW * D,), jnp.float32),
        pltpu.VMEM((RPW,), jnp.float32),
        pltpu.VMEM((RPW,), jnp.float32),
        pltpu.VMEM((NU,), jnp.int32),
        pltpu.VMEM((NU,), jnp.float32),
        pltpu.VMEM((NU,), jnp.float32),
        pltpu.VMEM((NB,), jnp.int32),
        pltpu.VMEM((NB,), jnp.int32),
        pltpu.VMEM((NB,), jnp.float32),
        pltpu.VMEM((NB,), jnp.float32),
        pltpu.VMEM((NB,), jnp.float32),
        pltpu.VMEM((NB,), jnp.float32),
        pltpu.SemaphoreType.DMA,
        pltpu.SemaphoreType.DMA,
        pltpu.SemaphoreType.DMA,
    ],
    compiler_params=pltpu.CompilerParams(needs_layout_passes=False),
)
def _pgm_loss_sc(x_hbm, y_hbm, uv_hbm, w0_hbm, w1_hbm, bv1_hbm, bv2_hbm,
                 w00_hbm, w01_hbm, w10_hbm, w11_hbm, out_hbm,
                 xv, yv, accv, uvv, w0v, w1v, bv1v, bv2v, w00v, w01v, w10v,
                 w11v, sem0, sem1, csem):
    _sc_body(x_hbm, y_hbm, uv_hbm, w0_hbm, w1_hbm, bv1_hbm, bv2_hbm,
             w00_hbm, w01_hbm, w10_hbm, w11_hbm, out_hbm,
             xv, yv, accv, uvv, w0v, w1v, bv1v, bv2v, w00v, w01v, w10v,
             w11v, sem0, sem1, csem)


def kernel(X, y, univariate_vars, univariate_weights_0, univariate_weights_1,
           bivariate_vars_1, bivariate_vars_2, bivariate_weights_00,
           bivariate_weights_01, bivariate_weights_10, bivariate_weights_11):
    return _pgm_loss_sc(X.reshape(-1), y[:, 0], univariate_vars,
                        univariate_weights_0, univariate_weights_1,
                        bivariate_vars_1, bivariate_vars_2,
                        bivariate_weights_00, bivariate_weights_01,
                        bivariate_weights_10, bivariate_weights_11)


# speedup vs baseline: 1.7192x; 1.0159x over previous
import functools

import jax
import jax.numpy as jnp
from jax import lax
from jax.experimental import pallas as pl
from jax.experimental.pallas import tpu as pltpu
from jax.experimental.pallas import tpu_sc as plsc

NC = 2
NS = 16
L = 16
NW = NC * NS

N_ROWS = 16384
D = 127
RPW = N_ROWS // NW
G = RPW // L
U = 4
NU = 16
NB = 24


def _splat(vec, i):
    return vec.at[jnp.full((L,), i, dtype=jnp.int32)].get(
        mode="promise_in_bounds")


def _sc_body(x_hbm, y_hbm, uv_hbm, w0_hbm, w1_hbm, bv1_hbm, bv2_hbm,
             w00_hbm, w01_hbm, w10_hbm, w11_hbm, out_hbm,
             xv, yv, accv, uvv, w0v, w1v, bv1v, bv2v, w00v, w01v, w10v,
             w11v, sem):
    wid = lax.axis_index("s") * NC + lax.axis_index("c")
    base = wid * RPW

    copies = [
        pltpu.async_copy(x_hbm.at[pl.ds(base * D, RPW * D)], xv, sem),
        pltpu.async_copy(y_hbm.at[pl.ds(base, RPW)], yv, sem),
        pltpu.async_copy(uv_hbm, uvv, sem),
        pltpu.async_copy(w0_hbm, w0v, sem),
        pltpu.async_copy(w1_hbm, w1v, sem),
        pltpu.async_copy(bv1_hbm, bv1v, sem),
        pltpu.async_copy(bv2_hbm, bv2v, sem),
        pltpu.async_copy(w00_hbm, w00v, sem),
        pltpu.async_copy(w01_hbm, w01v, sem),
        pltpu.async_copy(w10_hbm, w10v, sem),
        pltpu.async_copy(w11_hbm, w11v, sem),
    ]
    for c in copies:
        c.wait()

    lane = lax.iota(jnp.int32, L)

    uvec = uvv[...]
    avec = w1v[...] - w0v[...]

    def chunks(ref):
        return ref[pl.ds(0, L)], ref[pl.ds(8, L)]

    i1, i2 = chunks(bv1v)
    j1, j2 = chunks(bv2v)
    w00a, w00b = chunks(w00v)
    w01a, w01b = chunks(w01v)
    w10a, w10b = chunks(w10v)
    w11a, w11b = chunks(w11v)
    b1a, b1b = w10a - w00a, w10b - w00b
    b2a, b2b = w01a - w00a, w01b - w00b
    bba, bbb = w00a - w01a - w10a + w11a, w00b - w01b - w10b + w11b

    zeros = jnp.zeros((L,), jnp.float32)
    c0_parts = (w0v[...] + w00a + jnp.where(lane >= 8, w00b, zeros))
    c0 = jnp.sum(c0_parts)
    c0v = jnp.full((L,), c0, dtype=jnp.float32)

    def iter_body(it, _):
        g0 = it * U
        rowoffs = [(g0 + u) * (L * D) + lane * D for u in range(U)]
        ygs = [yv[pl.ds((g0 + u) * L, L)] for u in range(U)]
        accs = [c0v for _ in range(U)]

        for t in range(NU):
            idxs = _splat(uvec, t)
            cl = jnp.minimum(idxs, D - 1)
            isy = idxs == D
            at = _splat(avec, t)
            for u in range(U):
                tv = plsc.load_gather(xv, [rowoffs[u] + cl])
                tv = jnp.where(isy, ygs[u], tv)
                accs[u] = accs[u] + at * tv

        for t in range(NB):
            if t < 8:
                iv, jv = i1, j1
                b1, b2, bb = b1a, b2a, bba
                e = t
            else:
                iv, jv = i2, j2
                b1, b2, bb = b1b, b2b, bbb
                e = t - 8
            iis = _splat(iv, e)
            jjs = _splat(jv, e)
            cli = jnp.minimum(iis, D - 1)
            clj = jnp.minimum(jjs, D - 1)
            isyi = iis == D
            isyj = jjs == D
            b1t = _splat(b1, e)
            b2t = _splat(b2, e)
            bbt = _splat(bb, e)
            for u in range(U):
                t1 = plsc.load_gather(xv, [rowoffs[u] + cli])
                t1 = jnp.where(isyi, ygs[u], t1)
                t2 = plsc.load_gather(xv, [rowoffs[u] + clj])
                t2 = jnp.where(isyj, ygs[u], t2)
                accs[u] = accs[u] + t1 * (b1t + bbt * t2) + b2t * t2

        for u in range(U):
            accv[pl.ds((g0 + u) * L, L)] = accs[u]
        return _

    lax.fori_loop(0, G // U, iter_body, None)
    pltpu.sync_copy(accv, out_hbm.at[pl.ds(base, RPW)])


@functools.partial(
    pl.kernel,
    out_type=jax.ShapeDtypeStruct((N_ROWS,), jnp.float32),
    mesh=plsc.VectorSubcoreMesh(core_axis_name="c", subcore_axis_name="s",
                                num_cores=NC, num_subcores=NS),
    scratch_types=[
        pltpu.VMEM((RPW * D,), jnp.float32),
        pltpu.VMEM((RPW,), jnp.float32),
        pltpu.VMEM((RPW,), jnp.float32),
        pltpu.VMEM((NU,), jnp.int32),
        pltpu.VMEM((NU,), jnp.float32),
        pltpu.VMEM((NU,), jnp.float32),
        pltpu.VMEM((NB,), jnp.int32),
        pltpu.VMEM((NB,), jnp.int32),
        pltpu.VMEM((NB,), jnp.float32),
        pltpu.VMEM((NB,), jnp.float32),
        pltpu.VMEM((NB,), jnp.float32),
        pltpu.VMEM((NB,), jnp.float32),
        pltpu.SemaphoreType.DMA,
    ],
    compiler_params=pltpu.CompilerParams(needs_layout_passes=False),
)
def _pgm_loss_sc(x_hbm, y_hbm, uv_hbm, w0_hbm, w1_hbm, bv1_hbm, bv2_hbm,
                 w00_hbm, w01_hbm, w10_hbm, w11_hbm, out_hbm,
                 xv, yv, accv, uvv, w0v, w1v, bv1v, bv2v, w00v, w01v, w10v,
                 w11v, sem):
    _sc_body(x_hbm, y_hbm, uv_hbm, w0_hbm, w1_hbm, bv1_hbm, bv2_hbm,
             w00_hbm, w01_hbm, w10_hbm, w11_hbm, out_hbm,
             xv, yv, accv, uvv, w0v, w1v, bv1v, bv2v, w00v, w01v, w10v,
             w11v, sem)


def kernel(X, y, univariate_vars, univariate_weights_0, univariate_weights_1,
           bivariate_vars_1, bivariate_vars_2, bivariate_weights_00,
           bivariate_weights_01, bivariate_weights_10, bivariate_weights_11):
    return _pgm_loss_sc(X.reshape(-1), y[:, 0], univariate_vars,
                        univariate_weights_0, univariate_weights_1,
                        bivariate_vars_1, bivariate_vars_2,
                        bivariate_weights_00, bivariate_weights_01,
                        bivariate_weights_10, bivariate_weights_11)
